# baseline (speedup 1.0000x reference)
import functools

import jax
import jax.numpy as jnp
from jax import lax
from jax.experimental import pallas as pl
from jax.experimental.pallas import tpu as pltpu
from jax.experimental.pallas import tpu_sc as plsc

SEF = 64
SNF = 64
_NC = 2
_NS = 16
_NW = _NC * _NS
_CS = 128


def _gather_call(x, dsti, srci):
    nw, nch, cs = dsti.shape
    epw = nch * cs
    e_pad = nw * epw
    mesh = plsc.VectorSubcoreMesh(core_axis_name="c", subcore_axis_name="s")

    @functools.partial(
        pl.kernel, mesh=mesh,
        out_type=(jax.ShapeDtypeStruct((e_pad, 128), jnp.float32),
                  jax.ShapeDtypeStruct((e_pad, 128), jnp.float32)),
        scratch_types=[
            pltpu.VMEM((nch, cs), jnp.int32),
            pltpu.VMEM((nch, cs), jnp.int32),
            pltpu.VMEM((cs, 128), jnp.float32),
            pltpu.VMEM((cs, 128), jnp.float32),
            pltpu.SemaphoreType.DMA,
            pltpu.SemaphoreType.DMA,
        ],
    )
    def gather_k(x_hbm, dsti_hbm, srci_hbm, xi_hbm, xj_hbm,
                 dsti_v, srci_v, bufi_v, bufj_v, semi, semj):
        cid = lax.axis_index("c")
        sid = lax.axis_index("s")
        wid = cid * _NS + sid
        pltpu.sync_copy(dsti_hbm.at[wid], dsti_v)
        pltpu.sync_copy(srci_hbm.at[wid], srci_v)
        base = wid * epw

        def body(g, carry):
            off = base + g * cs
            ci = pltpu.async_copy(x_hbm.at[dsti_v.at[g]], bufi_v, semi)
            cj = pltpu.async_copy(x_hbm.at[srci_v.at[g]], bufj_v, semj)
            ci.wait()
            cj.wait()
            pltpu.sync_copy(bufi_v, xi_hbm.at[pl.ds(off, cs)])
            pltpu.sync_copy(bufj_v, xj_hbm.at[pl.ds(off, cs)])
            return carry

        lax.fori_loop(0, nch, body, 0)

    return gather_k(x, dsti, srci)


def _scatter_call(e, dsti, n_acc, n_pad):
    nw, nch, cs = dsti.shape
    epw = nch * cs
    rps = n_acc // _NS
    mesh = plsc.VectorSubcoreMesh(core_axis_name="c", subcore_axis_name="s")

    @functools.partial(
        pl.kernel, mesh=mesh,
        out_type=jax.ShapeDtypeStruct((_NC, n_pad, 128), jnp.float32),
        scratch_types=[
            pltpu.VMEM_SHARED((n_acc, 128), jnp.float32),
            pltpu.VMEM((cs, 128), jnp.float32),
            pltpu.VMEM((cs, 128), jnp.float32),
            pltpu.VMEM((8, cs), jnp.int32),
            pltpu.SemaphoreType.DMA,
            pltpu.SemaphoreType.DMA,
        ],
    )
    def scatter_k(e_hbm, dsti_hbm, out_hbm, acc_sh, u0, u1, idx8, sr0, sr1):
        cid = lax.axis_index("c")
        sid = lax.axis_index("s")
        wid = cid * _NS + sid

        def zrow(i, carry):
            for j in range(8):
                u0[i, pl.ds(j * 16, 16)] = jnp.zeros((16,), jnp.float32)
            return carry

        lax.fori_loop(0, cs, zrow, 0)
        base_r = pl.multiple_of(sid * rps, 8)
        for k in range(rps // cs):
            pltpu.sync_copy(u0, acc_sh.at[pl.ds(base_r + k * cs, cs)])
        rem = rps - (rps // cs) * cs
        if rem:
            pltpu.sync_copy(u0.at[pl.ds(0, rem)],
                            acc_sh.at[pl.ds(base_r + (rps // cs) * cs, rem)])
        plsc.subcore_barrier()
        base = wid * epw
        bufs = (u0, u1)
        sems = (sr0, sr1)

        def grp(k, carry):
            g0 = pl.multiple_of(8 * k, 8)
            pltpu.sync_copy(dsti_hbm.at[wid, pl.ds(g0, 8)], idx8)
            pend = pltpu.async_copy(
                e_hbm.at[pl.ds(base + g0 * cs, cs)], u0, sr0)
            for c in range(8):
                nxt = None
                if c < 7:
                    nxt = pltpu.async_copy(
                        e_hbm.at[pl.ds(base + (g0 + c + 1) * cs, cs)],
                        bufs[(c + 1) % 2], sems[(c + 1) % 2])
                pend.wait()
                pltpu.sync_copy(bufs[c % 2], acc_sh.at[idx8.at[c]], add=True)
                pend = nxt
            return carry

        lax.fori_loop(0, nch // 8, grp, 0)
        plsc.subcore_barrier()
        pltpu.sync_copy(acc_sh.at[pl.ds(base_r, rps)],
                        out_hbm.at[cid, pl.ds(base_r, rps)])

    return scatter_k(e, dsti)


def _dot(a, b):
    return jnp.dot(a, b, preferred_element_type=jnp.float32)


def _enc_node_body(nf, tf, emb, wa, wb, b0, w1, b1, x_ref):
    t = emb[0:1, :] + tf[...] * (emb[1:2, :] - emb[0:1, :])
    h = jnp.maximum(_dot(nf[...], wa[...]) + _dot(t, wb[...]) + b0[...], 0.0)
    x_ref[...] = _dot(h, w1[...]) + b1[...]


def _enc_edge_body(ef, w0, b0, w1, b1, out_ref):
    h = jnp.maximum(_dot(ef[...], w0[...]) + b0[...], 0.0)
    out_ref[...] = _dot(h, w1[...]) + b1[...]


def _edge_mlp_body(e_ref, xi_ref, xj_ref, wa, wb, wc, b0, w1, b1, out_ref):
    e = e_ref[...]
    z = (_dot(e[:, SEF:], wa[...]) + _dot(xi_ref[:, SNF:], wb[...])
         + _dot(xj_ref[:, SNF:], wc[...]) + b0[...])
    h = jnp.maximum(z, 0.0)
    out_ref[...] = e + _dot(h, w1[...]) + b1[...]


def _node_mlp_body(sp, x_ref, wa, wb, b0, w1, b1, out_ref):
    s = sp[0] + sp[1]
    inv = 1.0 / jnp.maximum(s[:, 0:1], 1.0)
    aggr = s[:, SEF:] * inv
    x = x_ref[...]
    h = jnp.maximum(_dot(aggr, wa[...]) + _dot(x[:, SNF:], wb[...])
                    + b0[...], 0.0)
    out_ref[...] = x + _dot(h, w1[...]) + b1[...]


def _dec_body(x_ref, w0, b0, w1, b1, out_ref):
    h = jnp.maximum(_dot(x_ref[...], w0[...]) + b0[...], 0.0)
    out_ref[...] = _dot(h, w1[...]) + b1[...]


def _wspec(shape):
    return pl.BlockSpec(shape, lambda i: (0,) * len(shape))


def _enc_node_call(nf_p, tf_p, emb_p, wa, wb, b0, w1, b1, r):
    n_pad = nf_p.shape[0]
    return pl.pallas_call(
        _enc_node_body,
        grid=(n_pad // r,),
        in_specs=[
            pl.BlockSpec((r, 128), lambda i: (i, 0)),
            pl.BlockSpec((r, 1), lambda i: (i, 0)),
            _wspec((8, 16)), _wspec((128, 128)), _wspec((16, 128)),
            _wspec((1, 128)), _wspec((128, 128)), _wspec((1, 128)),
        ],
        out_specs=pl.BlockSpec((r, 128), lambda i: (i, 0)),
        out_shape=jax.ShapeDtypeStruct((n_pad, 128), jnp.float32),
    )(nf_p, tf_p, emb_p, wa, wb, b0, w1, b1)


def _enc_edge_call(ef, w0, b0, w1, b1, r):
    e_pad = ef.shape[0]
    return pl.pallas_call(
        _enc_edge_body,
        grid=(e_pad // r,),
        in_specs=[
            pl.BlockSpec((r, 16), lambda i: (i, 0)),
            _wspec((16, 128)), _wspec((1, 128)),
            _wspec((128, 128)), _wspec((1, 128)),
        ],
        out_specs=pl.BlockSpec((r, 128), lambda i: (i, 0)),
        out_shape=jax.ShapeDtypeStruct((e_pad, 128), jnp.float32),
    )(ef, w0, b0, w1, b1)


def _edge_mlp_call(e, xi, xj, wa, wb, wc, b0, w1, b1, r):
    e_pad = e.shape[0]
    blk = pl.BlockSpec((r, 128), lambda i: (i, 0))
    return pl.pallas_call(
        _edge_mlp_body,
        grid=(e_pad // r,),
        in_specs=[
            blk, blk, blk,
            _wspec((64, 128)), _wspec((64, 128)), _wspec((64, 128)),
            _wspec((1, 128)), _wspec((128, 128)), _wspec((1, 128)),
        ],
        out_specs=blk,
        out_shape=jax.ShapeDtypeStruct((e_pad, 128), jnp.float32),
    )(e, xi, xj, wa, wb, wc, b0, w1, b1)


def _node_mlp_call(summ_p, x, wa, wb, b0, w1, b1, r):
    n_pad = x.shape[0]
    blk = pl.BlockSpec((r, 128), lambda i: (i, 0))
    return pl.pallas_call(
        _node_mlp_body,
        grid=(n_pad // r,),
        in_specs=[
            pl.BlockSpec((2, r, 128), lambda i: (0, i, 0)),
            blk,
            _wspec((64, 128)), _wspec((64, 128)),
            _wspec((1, 128)), _wspec((128, 128)), _wspec((1, 128)),
        ],
        out_specs=blk,
        out_shape=jax.ShapeDtypeStruct((n_pad, 128), jnp.float32),
    )(summ_p, x, wa, wb, b0, w1, b1)


def _dec_call(x, w0, b0, w1, b1, r):
    n_pad = x.shape[0]
    blk = pl.BlockSpec((r, 128), lambda i: (i, 0))
    return pl.pallas_call(
        _dec_body,
        grid=(n_pad // r,),
        in_specs=[blk, _wspec((128, 128)), _wspec((1, 128)),
                  _wspec((128, 128)), _wspec((1, 128))],
        out_specs=blk,
        out_shape=jax.ShapeDtypeStruct((n_pad, 128), jnp.float32),
    )(x, w0, b0, w1, b1)


def _upper_pad(w1, b1, carrier=0.0):
    w = jnp.concatenate([jnp.zeros((w1.shape[0], SEF), w1.dtype), w1], axis=1)
    b = jnp.concatenate(
        [jnp.full((1,), carrier, b1.dtype), jnp.zeros((SEF - 1,), b1.dtype),
         b1])[None, :]
    return w, b


def kernel(node_features, node_type, edge_index, edge_features, emb,
           enc_node_W, enc_node_b, enc_edge_W, enc_edge_b,
           proc_edge_W, proc_edge_b, proc_node_W, proc_node_b,
           dec_W, dec_b):
    n = node_features.shape[0]
    e_total = edge_features.shape[0]
    nr_steps = len(proc_edge_W)

    n_pad = (n // 128 + 1) * 128
    rn = n_pad // 16
    re = 512

    n_acc = n_pad
    n_spare = n_acc - n

    epw = e_total // _NW
    nch = -(-((epw + _CS - 1) // _CS) // 8) * 8
    epw_pad = nch * _CS
    e_pad = epw_pad * _NW
    padw = epw_pad - epw

    nf_p = jnp.pad(node_features, ((0, n_pad - n), (0, 0)))
    tf_p = jnp.pad(node_type.astype(jnp.float32)[:, None],
                   ((0, n_pad - n), (0, 0)))
    emb_p = jnp.pad(emb, ((0, 8 - emb.shape[0]), (0, 0)))
    ef_p = jnp.pad(edge_features.reshape(_NW, epw, -1),
                   ((0, 0), (0, padw), (0, 0))).reshape(e_pad, -1)
    src_pad = jnp.zeros((_NW, padw), jnp.int32)
    dst_pad = (n + (jnp.arange(_NW * padw, dtype=jnp.int32) % n_spare)
               ).reshape(_NW, padw)
    srci = jnp.concatenate(
        [edge_index[0].reshape(_NW, epw), src_pad], axis=1
    ).reshape(_NW, nch, _CS)
    dsti = jnp.concatenate(
        [edge_index[1].reshape(_NW, epw), dst_pad], axis=1
    ).reshape(_NW, nch, _CS)

    enW0, enW1 = enc_node_W
    enb0, enb1 = enc_node_b
    x = _enc_node_call(nf_p, tf_p, emb_p,
                       enW0[:128], enW0[128:], enb0[None, :],
                       enW1, enb1[None, :], rn)

    eeW0, eeW1 = enc_edge_W
    eeb0, eeb1 = enc_edge_b
    eW1p, eb1p = _upper_pad(eeW1[:, SEF:], eeb1[SEF:], carrier=1.0)
    e = _enc_edge_call(ef_p, eeW0, eeb0[None, :], eW1p, eb1p, re)

    for s in range(nr_steps):
        xi, xj = _gather_call(x, dsti, srci)
        peW0, peW1 = proc_edge_W[s]
        peb0, peb1 = proc_edge_b[s]
        pW1p, pb1p = _upper_pad(peW1, peb1)
        e = _edge_mlp_call(e, xi, xj,
                           peW0[:64], peW0[64:128], peW0[128:],
                           peb0[None, :], pW1p, pb1p, re)
        summ_p = _scatter_call(e, dsti, n_acc, n_pad)
        pnW0, pnW1 = proc_node_W[s]
        pnb0, pnb1 = proc_node_b[s]
        nW1p, nb1p = _upper_pad(pnW1, pnb1)
        x = _node_mlp_call(summ_p, x,
                           pnW0[:64], pnW0[64:128],
                           pnb0[None, :], nW1p, nb1p, rn)

    dW0, dW1 = dec_W
    db0, db1 = dec_b
    out_p = _dec_call(x, dW0, db0[None, :], dW1, db1[None, :], rn)
    return out_p[:n]


# speedup vs baseline: 1.2072x; 1.2072x over previous
import functools

import jax
import jax.numpy as jnp
from jax import lax
from jax.experimental import pallas as pl
from jax.experimental.pallas import tpu as pltpu
from jax.experimental.pallas import tpu_sc as plsc

SEF = 64
SNF = 64
_NC = 2
_NS = 16
_NW = _NC * _NS
_CS = 128


def _gather_call(x, dsti, srci):
    nw, nch, cs = dsti.shape
    epw = nch * cs
    e_pad = nw * epw
    mesh = plsc.VectorSubcoreMesh(core_axis_name="c", subcore_axis_name="s")

    @functools.partial(
        pl.kernel, mesh=mesh,
        out_type=(jax.ShapeDtypeStruct((e_pad, 128), jnp.float32),
                  jax.ShapeDtypeStruct((e_pad, 128), jnp.float32)),
        scratch_types=[
            pltpu.VMEM((nch, cs), jnp.int32),
            pltpu.VMEM((nch, cs), jnp.int32),
            pltpu.VMEM((cs, 128), jnp.float32),
            pltpu.VMEM((cs, 128), jnp.float32),
            pltpu.SemaphoreType.DMA,
            pltpu.SemaphoreType.DMA,
        ],
    )
    def gather_k(x_hbm, dsti_hbm, srci_hbm, xi_hbm, xj_hbm,
                 dsti_v, srci_v, bufi_v, bufj_v, semi, semj):
        cid = lax.axis_index("c")
        sid = lax.axis_index("s")
        wid = cid * _NS + sid
        pltpu.sync_copy(dsti_hbm.at[wid], dsti_v)
        pltpu.sync_copy(srci_hbm.at[wid], srci_v)
        base = wid * epw

        def body(g, carry):
            off = base + g * cs
            ci = pltpu.async_copy(x_hbm.at[dsti_v.at[g]], bufi_v, semi)
            cj = pltpu.async_copy(x_hbm.at[srci_v.at[g]], bufj_v, semj)
            ci.wait()
            cj.wait()
            pltpu.sync_copy(bufi_v, xi_hbm.at[pl.ds(off, cs)])
            pltpu.sync_copy(bufj_v, xj_hbm.at[pl.ds(off, cs)])
            return carry

        lax.fori_loop(0, nch, body, 0)

    return gather_k(x, dsti, srci)


def _scatter_call(e, dsti, n_acc, n_pad):
    nw, nch, cs = dsti.shape
    epw = nch * cs
    rps = n_acc // _NS
    mesh = plsc.VectorSubcoreMesh(core_axis_name="c", subcore_axis_name="s")

    @functools.partial(
        pl.kernel, mesh=mesh,
        out_type=jax.ShapeDtypeStruct((_NC, n_pad, 128), jnp.float32),
        scratch_types=[
            pltpu.VMEM_SHARED((n_acc, 128), jnp.float32),
            pltpu.VMEM((cs, 128), jnp.float32),
            pltpu.VMEM((cs, 128), jnp.float32),
            pltpu.VMEM((8, cs), jnp.int32),
            pltpu.SemaphoreType.DMA,
            pltpu.SemaphoreType.DMA,
        ],
    )
    def scatter_k(e_hbm, dsti_hbm, out_hbm, acc_sh, u0, u1, idx8, sr0, sr1):
        cid = lax.axis_index("c")
        sid = lax.axis_index("s")
        wid = cid * _NS + sid

        def zrow(i, carry):
            for j in range(8):
                u0[i, pl.ds(j * 16, 16)] = jnp.zeros((16,), jnp.float32)
            return carry

        lax.fori_loop(0, cs, zrow, 0)
        base_r = pl.multiple_of(sid * rps, 8)
        for k in range(rps // cs):
            pltpu.sync_copy(u0, acc_sh.at[pl.ds(base_r + k * cs, cs)])
        rem = rps - (rps // cs) * cs
        if rem:
            pltpu.sync_copy(u0.at[pl.ds(0, rem)],
                            acc_sh.at[pl.ds(base_r + (rps // cs) * cs, rem)])
        plsc.subcore_barrier()
        base = wid * epw
        bufs = (u0, u1)
        sems = (sr0, sr1)

        def grp(k, carry):
            g0 = pl.multiple_of(8 * k, 8)
            pltpu.sync_copy(dsti_hbm.at[wid, pl.ds(g0, 8)], idx8)
            pend = pltpu.async_copy(
                e_hbm.at[pl.ds(base + g0 * cs, cs)], u0, sr0)
            for c in range(8):
                nxt = None
                if c < 7:
                    nxt = pltpu.async_copy(
                        e_hbm.at[pl.ds(base + (g0 + c + 1) * cs, cs)],
                        bufs[(c + 1) % 2], sems[(c + 1) % 2])
                pend.wait()
                pltpu.sync_copy(bufs[c % 2], acc_sh.at[idx8.at[c]], add=True)
                pend = nxt
            return carry

        lax.fori_loop(0, nch // 8, grp, 0)
        plsc.subcore_barrier()
        pltpu.sync_copy(acc_sh.at[pl.ds(base_r, rps)],
                        out_hbm.at[cid, pl.ds(base_r, rps)])

    return scatter_k(e, dsti)


def _dot(a, b):
    return jnp.dot(a, b, preferred_element_type=jnp.float32)


def _enc_node_body(nf, tf, emb, wa, wb, b0, w1, b1, x_ref):
    t = emb[0:1, :] + tf[...] * (emb[1:2, :] - emb[0:1, :])
    h = jnp.maximum(_dot(nf[...], wa[...]) + _dot(t, wb[...]) + b0[...], 0.0)
    x_ref[...] = _dot(h, w1[...]) + b1[...]


def _enc_edge_body(ef, w0, b0, w1, b1, out_ref):
    h = jnp.maximum(_dot(ef[...], w0[...]) + b0[...], 0.0)
    out_ref[...] = _dot(h, w1[...]) + b1[...]


def _edge_mlp_body(e_ref, xi_ref, xj_ref, wa, wb, wc, b0, w1, b1, out_ref):
    e = e_ref[...]
    z = (_dot(e[:, SEF:], wa[...]) + _dot(xi_ref[:, SNF:], wb[...])
         + _dot(xj_ref[:, SNF:], wc[...]) + b0[...])
    h = jnp.maximum(z, 0.0)
    out_ref[...] = e + _dot(h, w1[...]) + b1[...]


def _node_mlp_body(sp, x_ref, wa, wb, b0, w1, b1, out_ref):
    s = sp[0] + sp[1]
    inv = 1.0 / jnp.maximum(s[:, 0:1], 1.0)
    aggr = s[:, SEF:] * inv
    x = x_ref[...]
    h = jnp.maximum(_dot(aggr, wa[...]) + _dot(x[:, SNF:], wb[...])
                    + b0[...], 0.0)
    out_ref[...] = x + _dot(h, w1[...]) + b1[...]


def _dec_body(x_ref, w0, b0, w1, b1, out_ref):
    h = jnp.maximum(_dot(x_ref[...], w0[...]) + b0[...], 0.0)
    out_ref[...] = _dot(h, w1[...]) + b1[...]


def _wspec(shape):
    return pl.BlockSpec(shape, lambda i: (0,) * len(shape))


def _enc_node_call(nf_p, tf_p, emb_p, wa, wb, b0, w1, b1, r):
    n_pad = nf_p.shape[0]
    return pl.pallas_call(
        _enc_node_body,
        grid=(n_pad // r,),
        in_specs=[
            pl.BlockSpec((r, 128), lambda i: (i, 0)),
            pl.BlockSpec((r, 1), lambda i: (i, 0)),
            _wspec((8, 16)), _wspec((128, 128)), _wspec((16, 128)),
            _wspec((1, 128)), _wspec((128, 128)), _wspec((1, 128)),
        ],
        out_specs=pl.BlockSpec((r, 128), lambda i: (i, 0)),
        out_shape=jax.ShapeDtypeStruct((n_pad, 128), jnp.float32),
    )(nf_p, tf_p, emb_p, wa, wb, b0, w1, b1)


def _enc_edge_call(ef, w0, b0, w1, b1, r):
    e_pad = ef.shape[0]
    return pl.pallas_call(
        _enc_edge_body,
        grid=(e_pad // r,),
        in_specs=[
            pl.BlockSpec((r, 16), lambda i: (i, 0)),
            _wspec((16, 128)), _wspec((1, 128)),
            _wspec((128, 128)), _wspec((1, 128)),
        ],
        out_specs=pl.BlockSpec((r, 128), lambda i: (i, 0)),
        out_shape=jax.ShapeDtypeStruct((e_pad, 128), jnp.float32),
    )(ef, w0, b0, w1, b1)


def _edge_mlp_call(e, xi, xj, wa, wb, wc, b0, w1, b1, r):
    e_pad = e.shape[0]
    blk = pl.BlockSpec((r, 128), lambda i: (i, 0))
    return pl.pallas_call(
        _edge_mlp_body,
        grid=(e_pad // r,),
        in_specs=[
            blk, blk, blk,
            _wspec((64, 128)), _wspec((64, 128)), _wspec((64, 128)),
            _wspec((1, 128)), _wspec((128, 128)), _wspec((1, 128)),
        ],
        out_specs=blk,
        out_shape=jax.ShapeDtypeStruct((e_pad, 128), jnp.float32),
    )(e, xi, xj, wa, wb, wc, b0, w1, b1)


def _node_mlp_call(summ_p, x, wa, wb, b0, w1, b1, r):
    n_pad = x.shape[0]
    blk = pl.BlockSpec((r, 128), lambda i: (i, 0))
    return pl.pallas_call(
        _node_mlp_body,
        grid=(n_pad // r,),
        in_specs=[
            pl.BlockSpec((2, r, 128), lambda i: (0, i, 0)),
            blk,
            _wspec((64, 128)), _wspec((64, 128)),
            _wspec((1, 128)), _wspec((128, 128)), _wspec((1, 128)),
        ],
        out_specs=blk,
        out_shape=jax.ShapeDtypeStruct((n_pad, 128), jnp.float32),
    )(summ_p, x, wa, wb, b0, w1, b1)


def _dec_call(x, w0, b0, w1, b1, r):
    n_pad = x.shape[0]
    blk = pl.BlockSpec((r, 128), lambda i: (i, 0))
    return pl.pallas_call(
        _dec_body,
        grid=(n_pad // r,),
        in_specs=[blk, _wspec((128, 128)), _wspec((1, 128)),
                  _wspec((128, 128)), _wspec((1, 128))],
        out_specs=blk,
        out_shape=jax.ShapeDtypeStruct((n_pad, 128), jnp.float32),
    )(x, w0, b0, w1, b1)


def _upper_pad(w1, b1, carrier=0.0):
    w = jnp.concatenate([jnp.zeros((w1.shape[0], SEF), w1.dtype), w1], axis=1)
    b = jnp.concatenate(
        [jnp.full((1,), carrier, b1.dtype), jnp.zeros((SEF - 1,), b1.dtype),
         b1])[None, :]
    return w, b


def kernel(node_features, node_type, edge_index, edge_features, emb,
           enc_node_W, enc_node_b, enc_edge_W, enc_edge_b,
           proc_edge_W, proc_edge_b, proc_node_W, proc_node_b,
           dec_W, dec_b):
    n = node_features.shape[0]
    e_total = edge_features.shape[0]
    nr_steps = len(proc_edge_W)

    n_pad = (n // 128 + 1) * 128
    rn = n_pad // 16
    re = 512

    n_acc = n_pad
    n_spare = n_acc - n

    epw = e_total // _NW
    nch = -(-((epw + _CS - 1) // _CS) // 8) * 8
    epw_pad = nch * _CS
    e_pad = epw_pad * _NW
    padw = epw_pad - epw

    nf_p = jnp.pad(node_features, ((0, n_pad - n), (0, 0)))
    tf_p = jnp.pad(node_type.astype(jnp.float32)[:, None],
                   ((0, n_pad - n), (0, 0)))
    emb_p = jnp.pad(emb, ((0, 8 - emb.shape[0]), (0, 0)))
    ef_p = jnp.pad(edge_features.reshape(_NW, epw, -1),
                   ((0, 0), (0, padw), (0, 0))).reshape(e_pad, -1)
    spread = (jnp.arange(_NW * padw, dtype=jnp.int32) * 131) % n
    gat_pad = spread.reshape(_NW, padw)
    dst_pad = (n + (jnp.arange(_NW * padw, dtype=jnp.int32) % n_spare)
               ).reshape(_NW, padw)
    srci = jnp.concatenate(
        [edge_index[0].reshape(_NW, epw), gat_pad], axis=1
    ).reshape(_NW, nch, _CS)
    dsti_g = jnp.concatenate(
        [edge_index[1].reshape(_NW, epw), gat_pad], axis=1
    ).reshape(_NW, nch, _CS)
    dsti = jnp.concatenate(
        [edge_index[1].reshape(_NW, epw), dst_pad], axis=1
    ).reshape(_NW, nch, _CS)

    enW0, enW1 = enc_node_W
    enb0, enb1 = enc_node_b
    x = _enc_node_call(nf_p, tf_p, emb_p,
                       enW0[:128], enW0[128:], enb0[None, :],
                       enW1, enb1[None, :], rn)

    eeW0, eeW1 = enc_edge_W
    eeb0, eeb1 = enc_edge_b
    eW1p, eb1p = _upper_pad(eeW1[:, SEF:], eeb1[SEF:], carrier=1.0)
    e = _enc_edge_call(ef_p, eeW0, eeb0[None, :], eW1p, eb1p, re)

    for s in range(nr_steps):
        xi, xj = _gather_call(x, dsti_g, srci)
        peW0, peW1 = proc_edge_W[s]
        peb0, peb1 = proc_edge_b[s]
        pW1p, pb1p = _upper_pad(peW1, peb1)
        e = _edge_mlp_call(e, xi, xj,
                           peW0[:64], peW0[64:128], peW0[128:],
                           peb0[None, :], pW1p, pb1p, re)
        summ_p = _scatter_call(e, dsti, n_acc, n_pad)
        pnW0, pnW1 = proc_node_W[s]
        pnb0, pnb1 = proc_node_b[s]
        nW1p, nb1p = _upper_pad(pnW1, pnb1)
        x = _node_mlp_call(summ_p, x,
                           pnW0[:64], pnW0[64:128],
                           pnb0[None, :], nW1p, nb1p, rn)

    dW0, dW1 = dec_W
    db0, db1 = dec_b
    out_p = _dec_call(x, dW0, db0[None, :], dW1, db1[None, :], rn)
    return out_p[:n]


# speedup vs baseline: 1.3092x; 1.0845x over previous
import functools

import jax
import jax.numpy as jnp
from jax import lax
from jax.experimental import pallas as pl
from jax.experimental.pallas import tpu as pltpu
from jax.experimental.pallas import tpu_sc as plsc

SEF = 64
SNF = 64
_NC = 2
_NS = 16
_NW = _NC * _NS
_CS = 128


def _gather_call(x, dsti, srci):
    nw, nch, cs = dsti.shape
    epw = nch * cs
    e_pad = nw * epw
    mesh = plsc.VectorSubcoreMesh(core_axis_name="c", subcore_axis_name="s")

    @functools.partial(
        pl.kernel, mesh=mesh,
        out_type=(jax.ShapeDtypeStruct((e_pad, 128), jnp.float32),
                  jax.ShapeDtypeStruct((e_pad, 128), jnp.float32)),
        scratch_types=[
            pltpu.VMEM((nch, cs), jnp.int32),
            pltpu.VMEM((nch, cs), jnp.int32),
            pltpu.VMEM((cs, 128), jnp.float32),
            pltpu.VMEM((cs, 128), jnp.float32),
            pltpu.SemaphoreType.DMA,
            pltpu.SemaphoreType.DMA,
        ],
    )
    def gather_k(x_hbm, dsti_hbm, srci_hbm, xi_hbm, xj_hbm,
                 dsti_v, srci_v, bufi_v, bufj_v, semi, semj):
        cid = lax.axis_index("c")
        sid = lax.axis_index("s")
        wid = cid * _NS + sid
        pltpu.sync_copy(dsti_hbm.at[wid], dsti_v)
        pltpu.sync_copy(srci_hbm.at[wid], srci_v)
        base = wid * epw

        def body(g, carry):
            off = base + g * cs
            ci = pltpu.async_copy(x_hbm.at[dsti_v.at[g]], bufi_v, semi)
            cj = pltpu.async_copy(x_hbm.at[srci_v.at[g]], bufj_v, semj)
            ci.wait()
            cj.wait()
            pltpu.sync_copy(bufi_v, xi_hbm.at[pl.ds(off, cs)])
            pltpu.sync_copy(bufj_v, xj_hbm.at[pl.ds(off, cs)])
            return carry

        lax.fori_loop(0, nch, body, 0)

    return gather_k(x, dsti, srci)


def _scatter_call(e, dsti, n_acc, n_pad):
    nw, nch, cs = dsti.shape
    epw = nch * cs
    rps = n_acc // _NS
    mesh = plsc.VectorSubcoreMesh(core_axis_name="c", subcore_axis_name="s")

    @functools.partial(
        pl.kernel, mesh=mesh,
        out_type=jax.ShapeDtypeStruct((_NC, n_pad, 128), jnp.float32),
        scratch_types=[
            pltpu.VMEM_SHARED((n_acc, 128), jnp.float32),
            pltpu.VMEM((cs, 128), jnp.float32),
            pltpu.VMEM((cs, 128), jnp.float32),
            pltpu.VMEM((8, cs), jnp.int32),
            pltpu.SemaphoreType.DMA,
            pltpu.SemaphoreType.DMA,
        ],
    )
    def scatter_k(e_hbm, dsti_hbm, out_hbm, acc_sh, u0, u1, idx8, sr0, sr1):
        cid = lax.axis_index("c")
        sid = lax.axis_index("s")
        wid = cid * _NS + sid

        def zrow(i, carry):
            for j in range(8):
                u0[i, pl.ds(j * 16, 16)] = jnp.zeros((16,), jnp.float32)
            return carry

        lax.fori_loop(0, cs, zrow, 0)
        base_r = pl.multiple_of(sid * rps, 8)
        for k in range(rps // cs):
            pltpu.sync_copy(u0, acc_sh.at[pl.ds(base_r + k * cs, cs)])
        rem = rps - (rps // cs) * cs
        if rem:
            pltpu.sync_copy(u0.at[pl.ds(0, rem)],
                            acc_sh.at[pl.ds(base_r + (rps // cs) * cs, rem)])
        plsc.subcore_barrier()
        base = wid * epw
        bufs = (u0, u1)
        sems = (sr0, sr1)

        def grp(k, carry):
            g0 = pl.multiple_of(8 * k, 8)
            pltpu.sync_copy(dsti_hbm.at[wid, pl.ds(g0, 8)], idx8)
            pend = pltpu.async_copy(
                e_hbm.at[pl.ds(base + g0 * cs, cs)], u0, sr0)
            for c in range(8):
                nxt = None
                if c < 7:
                    nxt = pltpu.async_copy(
                        e_hbm.at[pl.ds(base + (g0 + c + 1) * cs, cs)],
                        bufs[(c + 1) % 2], sems[(c + 1) % 2])
                pend.wait()
                pltpu.sync_copy(bufs[c % 2], acc_sh.at[idx8.at[c]], add=True)
                pend = nxt
            return carry

        lax.fori_loop(0, nch // 8, grp, 0)
        plsc.subcore_barrier()
        pltpu.sync_copy(acc_sh.at[pl.ds(base_r, rps)],
                        out_hbm.at[cid, pl.ds(base_r, rps)])

    return scatter_k(e, dsti)


def _dot(a, b):
    return jnp.dot(a, b, preferred_element_type=jnp.float32)


def _enc_node_body(nf, tf, emb, wa, wb, b0, w1, b1, x_ref):
    t = emb[0:1, :] + tf[...] * (emb[1:2, :] - emb[0:1, :])
    h = jnp.maximum(_dot(nf[...], wa[...]) + _dot(t, wb[...]) + b0[...], 0.0)
    x_ref[...] = _dot(h, w1[...]) + b1[...]


def _enc_edge_body(ef, w0, b0, w1, b1, out_ref):
    h = jnp.maximum(_dot(ef[...], w0[...]) + b0[...], 0.0)
    out_ref[...] = _dot(h, w1[...]) + b1[...]


def _edge_mlp_body(e_ref, xi_ref, xj_ref, wa, wb, wc, b0, w1, b1, out_ref):
    e = e_ref[...]
    z = (_dot(e[:, SEF:], wa[...]) + _dot(xi_ref[:, SNF:], wb[...])
         + _dot(xj_ref[:, SNF:], wc[...]) + b0[...])
    h = jnp.maximum(z, 0.0)
    out_ref[...] = e + _dot(h, w1[...]) + b1[...]


def _node_mlp_body(sp, sq, x_ref, wa, wb, b0, w1, b1, out_ref):
    s = (sp[0] + sp[1]) + (sq[0] + sq[1])
    inv = 1.0 / jnp.maximum(s[:, 0:1], 1.0)
    aggr = s[:, SEF:] * inv
    x = x_ref[...]
    h = jnp.maximum(_dot(aggr, wa[...]) + _dot(x[:, SNF:], wb[...])
                    + b0[...], 0.0)
    out_ref[...] = x + _dot(h, w1[...]) + b1[...]


def _dec_body(x_ref, w0, b0, w1, b1, out_ref):
    h = jnp.maximum(_dot(x_ref[...], w0[...]) + b0[...], 0.0)
    out_ref[...] = _dot(h, w1[...]) + b1[...]


def _wspec(shape):
    return pl.BlockSpec(shape, lambda i: (0,) * len(shape))


def _enc_node_call(nf_p, tf_p, emb_p, wa, wb, b0, w1, b1, r):
    n_pad = nf_p.shape[0]
    return pl.pallas_call(
        _enc_node_body,
        grid=(n_pad // r,),
        in_specs=[
            pl.BlockSpec((r, 128), lambda i: (i, 0)),
            pl.BlockSpec((r, 1), lambda i: (i, 0)),
            _wspec((8, 16)), _wspec((128, 128)), _wspec((16, 128)),
            _wspec((1, 128)), _wspec((128, 128)), _wspec((1, 128)),
        ],
        out_specs=pl.BlockSpec((r, 128), lambda i: (i, 0)),
        out_shape=jax.ShapeDtypeStruct((n_pad, 128), jnp.float32),
    )(nf_p, tf_p, emb_p, wa, wb, b0, w1, b1)


def _enc_edge_call(ef, w0, b0, w1, b1, r):
    e_pad = ef.shape[0]
    return pl.pallas_call(
        _enc_edge_body,
        grid=(e_pad // r,),
        in_specs=[
            pl.BlockSpec((r, 16), lambda i: (i, 0)),
            _wspec((16, 128)), _wspec((1, 128)),
            _wspec((128, 128)), _wspec((1, 128)),
        ],
        out_specs=pl.BlockSpec((r, 128), lambda i: (i, 0)),
        out_shape=jax.ShapeDtypeStruct((e_pad, 128), jnp.float32),
    )(ef, w0, b0, w1, b1)


def _edge_mlp_call(e, xi, xj, wa, wb, wc, b0, w1, b1, r):
    e_pad = e.shape[0]
    blk = pl.BlockSpec((r, 128), lambda i: (i, 0))
    return pl.pallas_call(
        _edge_mlp_body,
        grid=(e_pad // r,),
        in_specs=[
            blk, blk, blk,
            _wspec((64, 128)), _wspec((64, 128)), _wspec((64, 128)),
            _wspec((1, 128)), _wspec((128, 128)), _wspec((1, 128)),
        ],
        out_specs=blk,
        out_shape=jax.ShapeDtypeStruct((e_pad, 128), jnp.float32),
    )(e, xi, xj, wa, wb, wc, b0, w1, b1)


def _node_mlp_call(summ_p, summ_q, x, wa, wb, b0, w1, b1, r):
    n_pad = x.shape[0]
    blk = pl.BlockSpec((r, 128), lambda i: (i, 0))
    sblk = pl.BlockSpec((2, r, 128), lambda i: (0, i, 0))
    return pl.pallas_call(
        _node_mlp_body,
        grid=(n_pad // r,),
        in_specs=[
            sblk, sblk,
            blk,
            _wspec((64, 128)), _wspec((64, 128)),
            _wspec((1, 128)), _wspec((128, 128)), _wspec((1, 128)),
        ],
        out_specs=blk,
        out_shape=jax.ShapeDtypeStruct((n_pad, 128), jnp.float32),
    )(summ_p, summ_q, x, wa, wb, b0, w1, b1)


def _dec_call(x, w0, b0, w1, b1, r):
    n_pad = x.shape[0]
    blk = pl.BlockSpec((r, 128), lambda i: (i, 0))
    return pl.pallas_call(
        _dec_body,
        grid=(n_pad // r,),
        in_specs=[blk, _wspec((128, 128)), _wspec((1, 128)),
                  _wspec((128, 128)), _wspec((1, 128))],
        out_specs=blk,
        out_shape=jax.ShapeDtypeStruct((n_pad, 128), jnp.float32),
    )(x, w0, b0, w1, b1)


def _upper_pad(w1, b1, carrier=0.0):
    w = jnp.concatenate([jnp.zeros((w1.shape[0], SEF), w1.dtype), w1], axis=1)
    b = jnp.concatenate(
        [jnp.full((1,), carrier, b1.dtype), jnp.zeros((SEF - 1,), b1.dtype),
         b1])[None, :]
    return w, b


def kernel(node_features, node_type, edge_index, edge_features, emb,
           enc_node_W, enc_node_b, enc_edge_W, enc_edge_b,
           proc_edge_W, proc_edge_b, proc_node_W, proc_node_b,
           dec_W, dec_b):
    n = node_features.shape[0]
    e_total = edge_features.shape[0]
    nr_steps = len(proc_edge_W)

    n_pad = (n // 128 + 1) * 128
    rn = n_pad // 16
    re = 512

    n_acc = n_pad
    n_spare = n_acc - n

    epw = e_total // _NW
    nch = -(-((epw + _CS - 1) // _CS) // 8) * 8
    epw_pad = nch * _CS
    e_pad = epw_pad * _NW
    padw = epw_pad - epw

    nf_p = jnp.pad(node_features, ((0, n_pad - n), (0, 0)))
    tf_p = jnp.pad(node_type.astype(jnp.float32)[:, None],
                   ((0, n_pad - n), (0, 0)))
    emb_p = jnp.pad(emb, ((0, 8 - emb.shape[0]), (0, 0)))
    ef_p = jnp.pad(edge_features.reshape(_NW, epw, -1),
                   ((0, 0), (0, padw), (0, 0))).reshape(e_pad, -1)
    spread = (jnp.arange(_NW * padw, dtype=jnp.int32) * 131) % n
    gat_pad = spread.reshape(_NW, padw)
    dst_pad = (n + (jnp.arange(_NW * padw, dtype=jnp.int32) % n_spare)
               ).reshape(_NW, padw)
    srci = jnp.concatenate(
        [edge_index[0].reshape(_NW, epw), gat_pad], axis=1
    ).reshape(_NW, nch, _CS)
    dsti_g = jnp.concatenate(
        [edge_index[1].reshape(_NW, epw), gat_pad], axis=1
    ).reshape(_NW, nch, _CS)
    dsti = jnp.concatenate(
        [edge_index[1].reshape(_NW, epw), dst_pad], axis=1
    ).reshape(_NW, nch, _CS)

    enW0, enW1 = enc_node_W
    enb0, enb1 = enc_node_b
    x = _enc_node_call(nf_p, tf_p, emb_p,
                       enW0[:128], enW0[128:], enb0[None, :],
                       enW1, enb1[None, :], rn)

    eeW0, eeW1 = enc_edge_W
    eeb0, eeb1 = enc_edge_b
    eW1p, eb1p = _upper_pad(eeW1[:, SEF:], eeb1[SEF:], carrier=1.0)

    hch = nch // 2
    ef3 = ef_p.reshape(_NW, epw_pad, -1)
    ef_h = (ef3[:, :hch * _CS].reshape(-1, ef3.shape[-1]),
            ef3[:, hch * _CS:].reshape(-1, ef3.shape[-1]))
    srci_h = (srci[:, :hch], srci[:, hch:])
    dsti_g_h = (dsti_g[:, :hch], dsti_g[:, hch:])
    dsti_h = (dsti[:, :hch], dsti[:, hch:])

    e_h = [_enc_edge_call(ef_h[h], eeW0, eeb0[None, :], eW1p, eb1p, re)
           for h in range(2)]

    for s in range(nr_steps):
        peW0, peW1 = proc_edge_W[s]
        peb0, peb1 = proc_edge_b[s]
        pW1p, pb1p = _upper_pad(peW1, peb1)
        gath = [_gather_call(x, dsti_g_h[h], srci_h[h]) for h in range(2)]
        summ = []
        for h in range(2):
            xi, xj = gath[h]
            e_h[h] = _edge_mlp_call(e_h[h], xi, xj,
                                    peW0[:64], peW0[64:128], peW0[128:],
                                    peb0[None, :], pW1p, pb1p, re)
            summ.append(_scatter_call(e_h[h], dsti_h[h], n_acc, n_pad))
        pnW0, pnW1 = proc_node_W[s]
        pnb0, pnb1 = proc_node_b[s]
        nW1p, nb1p = _upper_pad(pnW1, pnb1)
        x = _node_mlp_call(summ[0], summ[1], x,
                           pnW0[:64], pnW0[64:128],
                           pnb0[None, :], nW1p, nb1p, rn)

    dW0, dW1 = dec_W
    db0, db1 = dec_b
    out_p = _dec_call(x, dW0, db0[None, :], dW1, db1[None, :], rn)
    return out_p[:n]


# speedup vs baseline: 1.3820x; 1.0556x over previous
import functools

import jax
import jax.numpy as jnp
from jax import lax
from jax.experimental import pallas as pl
from jax.experimental.pallas import tpu as pltpu
from jax.experimental.pallas import tpu_sc as plsc

SEF = 64
SNF = 64
_NC = 2
_NS = 16
_NW = _NC * _NS
_CS = 128


def _gather_call(x, dsti, srci):
    nw, nch, cs = dsti.shape
    epw = nch * cs
    e_pad = nw * epw
    mesh = plsc.VectorSubcoreMesh(core_axis_name="c", subcore_axis_name="s")

    @functools.partial(
        pl.kernel, mesh=mesh,
        out_type=(jax.ShapeDtypeStruct((e_pad, 128), jnp.float32),
                  jax.ShapeDtypeStruct((e_pad, 128), jnp.float32)),
        scratch_types=[
            pltpu.VMEM((nch, cs), jnp.int32),
            pltpu.VMEM((nch, cs), jnp.int32),
            pltpu.VMEM((cs, 128), jnp.float32),
            pltpu.VMEM((cs, 128), jnp.float32),
            pltpu.SemaphoreType.DMA,
            pltpu.SemaphoreType.DMA,
        ],
    )
    def gather_k(x_hbm, dsti_hbm, srci_hbm, xi_hbm, xj_hbm,
                 dsti_v, srci_v, bufi_v, bufj_v, semi, semj):
        cid = lax.axis_index("c")
        sid = lax.axis_index("s")
        wid = cid * _NS + sid
        pltpu.sync_copy(dsti_hbm.at[wid], dsti_v)
        pltpu.sync_copy(srci_hbm.at[wid], srci_v)
        base = wid * epw

        def body(g, carry):
            off = base + g * cs
            ci = pltpu.async_copy(x_hbm.at[dsti_v.at[g]], bufi_v, semi)
            cj = pltpu.async_copy(x_hbm.at[srci_v.at[g]], bufj_v, semj)
            ci.wait()
            cj.wait()
            pltpu.sync_copy(bufi_v, xi_hbm.at[pl.ds(off, cs)])
            pltpu.sync_copy(bufj_v, xj_hbm.at[pl.ds(off, cs)])
            return carry

        lax.fori_loop(0, nch, body, 0)

    return gather_k(x, dsti, srci)


def _scatter_call(e, dsti, n_acc, n_pad):
    nw, nch, cs = dsti.shape
    epw = nch * cs
    rps = n_acc // _NS
    mesh = plsc.VectorSubcoreMesh(core_axis_name="c", subcore_axis_name="s")

    @functools.partial(
        pl.kernel, mesh=mesh,
        out_type=jax.ShapeDtypeStruct((_NC, n_pad, 128), jnp.float32),
        scratch_types=[
            pltpu.VMEM_SHARED((n_acc, 128), jnp.float32),
            pltpu.VMEM((cs, 128), jnp.float32),
            pltpu.VMEM((cs, 128), jnp.float32),
            pltpu.VMEM((8, cs), jnp.int32),
            pltpu.SemaphoreType.DMA,
            pltpu.SemaphoreType.DMA,
        ],
    )
    def scatter_k(e_hbm, dsti_hbm, out_hbm, acc_sh, u0, u1, idx8, sr0, sr1):
        cid = lax.axis_index("c")
        sid = lax.axis_index("s")
        wid = cid * _NS + sid

        def zrow(i, carry):
            for j in range(8):
                u0[i, pl.ds(j * 16, 16)] = jnp.zeros((16,), jnp.float32)
            return carry

        lax.fori_loop(0, cs, zrow, 0)
        base_r = pl.multiple_of(sid * rps, 8)
        for k in range(rps // cs):
            pltpu.sync_copy(u0, acc_sh.at[pl.ds(base_r + k * cs, cs)])
        rem = rps - (rps // cs) * cs
        if rem:
            pltpu.sync_copy(u0.at[pl.ds(0, rem)],
                            acc_sh.at[pl.ds(base_r + (rps // cs) * cs, rem)])
        plsc.subcore_barrier()
        base = wid * epw
        bufs = (u0, u1)
        sems = (sr0, sr1)

        def grp(k, carry):
            g0 = pl.multiple_of(8 * k, 8)
            pltpu.sync_copy(dsti_hbm.at[wid, pl.ds(g0, 8)], idx8)
            pend = pltpu.async_copy(
                e_hbm.at[pl.ds(base + g0 * cs, cs)], u0, sr0)
            for c in range(8):
                nxt = None
                if c < 7:
                    nxt = pltpu.async_copy(
                        e_hbm.at[pl.ds(base + (g0 + c + 1) * cs, cs)],
                        bufs[(c + 1) % 2], sems[(c + 1) % 2])
                pend.wait()
                pltpu.sync_copy(bufs[c % 2], acc_sh.at[idx8.at[c]], add=True)
                pend = nxt
            return carry

        lax.fori_loop(0, nch // 8, grp, 0)
        plsc.subcore_barrier()
        pltpu.sync_copy(acc_sh.at[pl.ds(base_r, rps)],
                        out_hbm.at[cid, pl.ds(base_r, rps)])

    return scatter_k(e, dsti)


def _dot(a, b):
    return jnp.dot(a, b, preferred_element_type=jnp.float32)


def _enc_node_body(nf, tf, emb, wa, wb, b0, w1, b1, x_ref):
    t = emb[0:1, :] + tf[...] * (emb[1:2, :] - emb[0:1, :])
    h = jnp.maximum(_dot(nf[...], wa[...]) + _dot(t, wb[...]) + b0[...], 0.0)
    x_ref[...] = _dot(h, w1[...]) + b1[...]


def _enc_edge_body(ef, w0, b0, w1, b1, out_ref):
    h = jnp.maximum(_dot(ef[...], w0[...]) + b0[...], 0.0)
    out_ref[...] = _dot(h, w1[...]) + b1[...]


def _edge_mlp_body(e_ref, xi_ref, xj_ref, wa, wb, wc, b0, w1, b1, out_ref):
    e = e_ref[...]
    z = (_dot(e[:, SEF:], wa[...]) + _dot(xi_ref[:, SNF:], wb[...])
         + _dot(xj_ref[:, SNF:], wc[...]) + b0[...])
    h = jnp.maximum(z, 0.0)
    out_ref[...] = e + _dot(h, w1[...]) + b1[...]


def _make_node_mlp_body(k):
    def body(*refs):
        sps = refs[:k]
        x_ref, wa, wb, b0, w1, b1, out_ref = refs[k:]
        s = sps[0][0] + sps[0][1]
        for t in range(1, k):
            s = s + sps[t][0] + sps[t][1]
        inv = 1.0 / jnp.maximum(s[:, 0:1], 1.0)
        aggr = s[:, SEF:] * inv
        x = x_ref[...]
        h = jnp.maximum(_dot(aggr, wa[...]) + _dot(x[:, SNF:], wb[...])
                        + b0[...], 0.0)
        out_ref[...] = x + _dot(h, w1[...]) + b1[...]
    return body


def _dec_body(x_ref, w0, b0, w1, b1, out_ref):
    h = jnp.maximum(_dot(x_ref[...], w0[...]) + b0[...], 0.0)
    out_ref[...] = _dot(h, w1[...]) + b1[...]


def _wspec(shape):
    return pl.BlockSpec(shape, lambda i: (0,) * len(shape))


def _enc_node_call(nf_p, tf_p, emb_p, wa, wb, b0, w1, b1, r):
    n_pad = nf_p.shape[0]
    return pl.pallas_call(
        _enc_node_body,
        grid=(n_pad // r,),
        in_specs=[
            pl.BlockSpec((r, 128), lambda i: (i, 0)),
            pl.BlockSpec((r, 1), lambda i: (i, 0)),
            _wspec((8, 16)), _wspec((128, 128)), _wspec((16, 128)),
            _wspec((1, 128)), _wspec((128, 128)), _wspec((1, 128)),
        ],
        out_specs=pl.BlockSpec((r, 128), lambda i: (i, 0)),
        out_shape=jax.ShapeDtypeStruct((n_pad, 128), jnp.float32),
    )(nf_p, tf_p, emb_p, wa, wb, b0, w1, b1)


def _enc_edge_call(ef, w0, b0, w1, b1, r):
    e_pad = ef.shape[0]
    return pl.pallas_call(
        _enc_edge_body,
        grid=(e_pad // r,),
        in_specs=[
            pl.BlockSpec((r, 16), lambda i: (i, 0)),
            _wspec((16, 128)), _wspec((1, 128)),
            _wspec((128, 128)), _wspec((1, 128)),
        ],
        out_specs=pl.BlockSpec((r, 128), lambda i: (i, 0)),
        out_shape=jax.ShapeDtypeStruct((e_pad, 128), jnp.float32),
    )(ef, w0, b0, w1, b1)


def _edge_mlp_call(e, xi, xj, wa, wb, wc, b0, w1, b1, r):
    e_pad = e.shape[0]
    blk = pl.BlockSpec((r, 128), lambda i: (i, 0))
    return pl.pallas_call(
        _edge_mlp_body,
        grid=(e_pad // r,),
        in_specs=[
            blk, blk, blk,
            _wspec((64, 128)), _wspec((64, 128)), _wspec((64, 128)),
            _wspec((1, 128)), _wspec((128, 128)), _wspec((1, 128)),
        ],
        out_specs=blk,
        out_shape=jax.ShapeDtypeStruct((e_pad, 128), jnp.float32),
    )(e, xi, xj, wa, wb, wc, b0, w1, b1)


def _node_mlp_call(summs, x, wa, wb, b0, w1, b1, r):
    n_pad = x.shape[0]
    k = len(summs)
    blk = pl.BlockSpec((r, 128), lambda i: (i, 0))
    sblk = pl.BlockSpec((2, r, 128), lambda i: (0, i, 0))
    return pl.pallas_call(
        _make_node_mlp_body(k),
        grid=(n_pad // r,),
        in_specs=[sblk] * k + [
            blk,
            _wspec((64, 128)), _wspec((64, 128)),
            _wspec((1, 128)), _wspec((128, 128)), _wspec((1, 128)),
        ],
        out_specs=blk,
        out_shape=jax.ShapeDtypeStruct((n_pad, 128), jnp.float32),
    )(*summs, x, wa, wb, b0, w1, b1)


def _dec_call(x, w0, b0, w1, b1, r):
    n_pad = x.shape[0]
    blk = pl.BlockSpec((r, 128), lambda i: (i, 0))
    return pl.pallas_call(
        _dec_body,
        grid=(n_pad // r,),
        in_specs=[blk, _wspec((128, 128)), _wspec((1, 128)),
                  _wspec((128, 128)), _wspec((1, 128))],
        out_specs=blk,
        out_shape=jax.ShapeDtypeStruct((n_pad, 128), jnp.float32),
    )(x, w0, b0, w1, b1)


def _upper_pad(w1, b1, carrier=0.0):
    w = jnp.concatenate([jnp.zeros((w1.shape[0], SEF), w1.dtype), w1], axis=1)
    b = jnp.concatenate(
        [jnp.full((1,), carrier, b1.dtype), jnp.zeros((SEF - 1,), b1.dtype),
         b1])[None, :]
    return w, b


def kernel(node_features, node_type, edge_index, edge_features, emb,
           enc_node_W, enc_node_b, enc_edge_W, enc_edge_b,
           proc_edge_W, proc_edge_b, proc_node_W, proc_node_b,
           dec_W, dec_b):
    n = node_features.shape[0]
    e_total = edge_features.shape[0]
    nr_steps = len(proc_edge_W)

    n_pad = (n // 128 + 1) * 128
    rn = n_pad // 16
    re = 512

    n_acc = n_pad
    n_spare = n_acc - n

    epw = e_total // _NW
    nch = -(-((epw + _CS - 1) // _CS) // 8) * 8
    epw_pad = nch * _CS
    e_pad = epw_pad * _NW
    padw = epw_pad - epw

    nf_p = jnp.pad(node_features, ((0, n_pad - n), (0, 0)))
    tf_p = jnp.pad(node_type.astype(jnp.float32)[:, None],
                   ((0, n_pad - n), (0, 0)))
    emb_p = jnp.pad(emb, ((0, 8 - emb.shape[0]), (0, 0)))
    ef_p = jnp.pad(edge_features.reshape(_NW, epw, -1),
                   ((0, 0), (0, padw), (0, 0))).reshape(e_pad, -1)
    spread = (jnp.arange(_NW * padw, dtype=jnp.int32) * 131) % n
    gat_pad = spread.reshape(_NW, padw)
    dst_pad = (n + (jnp.arange(_NW * padw, dtype=jnp.int32) % n_spare)
               ).reshape(_NW, padw)
    srci = jnp.concatenate(
        [edge_index[0].reshape(_NW, epw), gat_pad], axis=1
    ).reshape(_NW, nch, _CS)
    dsti_g = jnp.concatenate(
        [edge_index[1].reshape(_NW, epw), gat_pad], axis=1
    ).reshape(_NW, nch, _CS)
    dsti = jnp.concatenate(
        [edge_index[1].reshape(_NW, epw), dst_pad], axis=1
    ).reshape(_NW, nch, _CS)

    enW0, enW1 = enc_node_W
    enb0, enb1 = enc_node_b
    x = _enc_node_call(nf_p, tf_p, emb_p,
                       enW0[:128], enW0[128:], enb0[None, :],
                       enW1, enb1[None, :], rn)

    eeW0, eeW1 = enc_edge_W
    eeb0, eeb1 = enc_edge_b
    eW1p, eb1p = _upper_pad(eeW1[:, SEF:], eeb1[SEF:], carrier=1.0)

    splits = [16, 24, 24, 16] if nch == 80 else [nch // 2, nch - nch // 2]
    bounds = [sum(splits[:i]) for i in range(len(splits) + 1)]
    ns = len(splits)
    ef3 = ef_p.reshape(_NW, epw_pad, -1)
    ef_h = [ef3[:, bounds[h] * _CS:bounds[h + 1] * _CS
                ].reshape(-1, ef3.shape[-1]) for h in range(ns)]
    srci_h = [srci[:, bounds[h]:bounds[h + 1]] for h in range(ns)]
    dsti_g_h = [dsti_g[:, bounds[h]:bounds[h + 1]] for h in range(ns)]
    dsti_h = [dsti[:, bounds[h]:bounds[h + 1]] for h in range(ns)]

    e_h = [_enc_edge_call(ef_h[h], eeW0, eeb0[None, :], eW1p, eb1p, re)
           for h in range(ns)]

    for s in range(nr_steps):
        peW0, peW1 = proc_edge_W[s]
        peb0, peb1 = proc_edge_b[s]
        pW1p, pb1p = _upper_pad(peW1, peb1)
        gath = [_gather_call(x, dsti_g_h[h], srci_h[h]) for h in range(ns)]
        summ = []
        for h in range(ns):
            xi, xj = gath[h]
            e_h[h] = _edge_mlp_call(e_h[h], xi, xj,
                                    peW0[:64], peW0[64:128], peW0[128:],
                                    peb0[None, :], pW1p, pb1p, re)
            summ.append(_scatter_call(e_h[h], dsti_h[h], n_acc, n_pad))
        pnW0, pnW1 = proc_node_W[s]
        pnb0, pnb1 = proc_node_b[s]
        nW1p, nb1p = _upper_pad(pnW1, pnb1)
        x = _node_mlp_call(summ, x,
                           pnW0[:64], pnW0[64:128],
                           pnb0[None, :], nW1p, nb1p, rn)

    dW0, dW1 = dec_W
    db0, db1 = dec_b
    out_p = _dec_call(x, dW0, db0[None, :], dW1, db1[None, :], rn)
    return out_p[:n]
